# baseline (speedup 1.0000x reference)
import functools

import jax
import jax.numpy as jnp
from jax import lax
from jax.experimental import pallas as pl
from jax.experimental.pallas import tpu as pltpu
from jax.experimental.pallas import tpu_sc as plsc

NC = 2
NS = 16
NW = NC * NS
L = 16


def _tile_body(b_per_w, d, uids_hbm, iids_hbm, utab_hbm, itab_hbm, out_hbm,
               uidx_v, iidx_v, urows_v, irows_v, out_v, sem_u, sem_i):
    wid = lax.axis_index("s") * NC + lax.axis_index("c")
    base = pl.multiple_of(wid * b_per_w, 8)

    pltpu.sync_copy(uids_hbm.at[pl.ds(base, b_per_w)], uidx_v)
    pltpu.sync_copy(iids_hbm.at[pl.ds(base, b_per_w)], iidx_v)
    cu = pltpu.async_copy(utab_hbm.at[uidx_v], urows_v, sem_u)
    ci = pltpu.async_copy(itab_hbm.at[iidx_v], irows_v, sem_i)
    cu.wait()
    ci.wait()

    lanes = lax.iota(jnp.int32, L)

    def group(g, _):
        rows = jnp.full((L,), g * L, jnp.int32) + lanes
        acc = jnp.zeros((L,), jnp.float32)
        for col in range(d):
            dcol = jnp.full((L,), col, jnp.int32)
            u = plsc.load_gather(urows_v, [rows, dcol])
            it = plsc.load_gather(irows_v, [rows, dcol])
            acc = acc + u * it
        prob = 1.0 / (1.0 + jnp.exp(-acc))
        out_v[pl.ds(pl.multiple_of(g * L, 8), L)] = prob
        return 0

    lax.fori_loop(0, b_per_w // L, group, 0)
    pltpu.sync_copy(out_v, out_hbm.at[pl.ds(base, b_per_w)])


def kernel(user_ids, item_ids, user_table, item_table):
    b = user_ids.shape[0]
    d = user_table.shape[1]
    b_per_w = b // NW

    run = pl.kernel(
        functools.partial(_tile_body, b_per_w, d),
        out_type=jax.ShapeDtypeStruct((b,), jnp.float32),
        mesh=plsc.VectorSubcoreMesh(core_axis_name="c", subcore_axis_name="s"),
        compiler_params=pltpu.CompilerParams(
            needs_layout_passes=False, use_tc_tiling_on_sc=False),
        scratch_types=[
            pltpu.VMEM((b_per_w,), jnp.int32),
            pltpu.VMEM((b_per_w,), jnp.int32),
            pltpu.VMEM((b_per_w, d), jnp.float32),
            pltpu.VMEM((b_per_w, d), jnp.float32),
            pltpu.VMEM((b_per_w,), jnp.float32),
            pltpu.SemaphoreType.DMA,
            pltpu.SemaphoreType.DMA,
        ],
    )
    return run(user_ids, item_ids, user_table, item_table)
